# Initial kernel scaffold; baseline (speedup 1.0000x reference)
import jax
import jax.numpy as jnp
from jax.experimental import pallas as pl


def kernel(x, edge_index, params):
    raise NotImplementedError("write your pallas kernel here")


# speedup vs baseline: 1.0001x; 1.0001x over previous
import functools

import jax
import jax.numpy as jnp
from jax.experimental import pallas as pl
from jax.experimental.pallas import tpu as pltpu

N_NODES = 10000
N_EDGES = 320000
H = 128
HEADS = 8
DH = 16

_BN_SCALE = 1.0 / (1.0 + 1e-5) ** 0.5

ROWS_BLK = 2000


def _prelude_body(x_ref, w_ref, b_ref, g_ref, bb_ref, o_ref):
    h = jnp.dot(x_ref[...], w_ref[...], preferred_element_type=jnp.float32)
    h = (h + b_ref[...]) * (_BN_SCALE * g_ref[...]) + bb_ref[...]
    o_ref[...] = jnp.maximum(h, 0.0)


def _prelude(x, w, b, g, bb):
    grid = (N_NODES // ROWS_BLK,)
    return pl.pallas_call(
        _prelude_body,
        grid=grid,
        in_specs=[
            pl.BlockSpec((ROWS_BLK, H), lambda i: (i, 0)),
            pl.BlockSpec((H, H), lambda i: (0, 0)),
            pl.BlockSpec((1, H), lambda i: (0, 0)),
            pl.BlockSpec((1, H), lambda i: (0, 0)),
            pl.BlockSpec((1, H), lambda i: (0, 0)),
        ],
        out_specs=pl.BlockSpec((ROWS_BLK, H), lambda i: (i, 0)),
        out_shape=jax.ShapeDtypeStruct((N_NODES, H), jnp.float32),
    )(x, w, b.reshape(1, H), g.reshape(1, H), bb.reshape(1, H))


def _ln(h, g, b):
    m = h.mean(-1, keepdims=True)
    v = h.var(-1, keepdims=True)
    return (h - m) / jnp.sqrt(v + 1e-5) * g + b


def _bn(h, g, b):
    return h * (_BN_SCALE * g) + b


def _gcn(x, src, dst, W, b):
    N = x.shape[0]
    deg = jnp.zeros((N,), x.dtype).at[dst].add(1.0)
    dinv = jax.lax.rsqrt(jnp.maximum(deg, 1.0))
    h = x @ W
    coef = (dinv[src] * dinv[dst])[:, None]
    out = jnp.zeros_like(h).at[dst].add(h[src] * coef)
    return out + b


def _gat(x, src, dst, W, a_s, a_d, b):
    N = x.shape[0]
    h = (x @ W).reshape(N, HEADS, DH)
    als = (h * a_s[None]).sum(-1)
    ald = (h * a_d[None]).sum(-1)
    e = jax.nn.leaky_relu(als[src] + ald[dst], 0.2)
    emax = jax.ops.segment_max(e, dst, num_segments=N)
    ee = jnp.exp(e - emax[dst])
    den = jax.ops.segment_sum(ee, dst, num_segments=N)
    alpha = ee / (den[dst] + 1e-16)
    out = jax.ops.segment_sum(h[src] * alpha[:, :, None], dst, num_segments=N)
    return out.reshape(N, H) + b


def kernel(x, edge_index, params):
    P = params
    loop = jnp.arange(x.shape[0], dtype=edge_index.dtype)
    src = jnp.concatenate([edge_index[0], loop])
    dst = jnp.concatenate([edge_index[1], loop])

    x1 = _prelude(x, P['np_W'], P['np_b'], P['np_bn_g'], P['np_bn_b'])
    h = _gcn(x1, src, dst, P['gcn_W'], P['gcn_b'])
    h = _ln(h, P['ln0_g'], P['ln0_b']) + x1
    xx = jax.nn.relu(h)
    for i in range(1, 4):
        h = _gat(xx, src, dst, P['gat%d_W' % i], P['gat%d_as' % i],
                 P['gat%d_ad' % i], P['gat%d_b' % i])
        h = _ln(h, P['ln%d_g' % i], P['ln%d_b' % i]) + xx
        xx = jax.nn.relu(h)
    pw = jax.nn.softmax(P['pool_w'])
    gm = xx.mean(0, keepdims=True) * pw[0]
    gx = xx.max(0, keepdims=True) * pw[1]
    ga = xx.sum(0, keepdims=True) * pw[2]
    gf = jnp.concatenate([gm, gx, ga], axis=1)
    ctx = xx.mean(0, keepdims=True)
    ctx = jax.nn.relu(ctx @ P['c1_W'] + P['c1_b'])
    ctx = ctx @ P['c2_W'] + P['c2_b']
    ctx = jnp.broadcast_to(ctx, (xx.shape[0], ctx.shape[1]))
    nwc = jnp.concatenate([xx, ctx], axis=1)
    s = jax.nn.relu(_bn(nwc @ P['s1_W'] + P['s1_b'], P['s1_bn_g'], P['s1_bn_b']))
    s = jax.nn.relu(s @ P['s2_W'] + P['s2_b'])
    s = s @ P['s3_W'] + P['s3_b']
    scores = jax.nn.sigmoid(s[:, 0])
    c = jax.nn.relu(_bn(gf @ P['h1_W'] + P['h1_b'], P['h1_bn_g'], P['h1_bn_b']))
    c = jax.nn.relu(_bn(c @ P['h2_W'] + P['h2_b'], P['h2_bn_g'], P['h2_bn_b']))
    c = jax.nn.relu(c @ P['h3_W'] + P['h3_b'])
    logits = c @ P['h4_W'] + P['h4_b']
    fp = jax.nn.relu(xx @ P['p1_W'] + P['p1_b'])
    fp = fp @ P['p2_W'] + P['p2_b']
    return logits, scores, gf, xx, fp


# speedup vs baseline: 41.6931x; 41.6908x over previous
import functools

import jax
import jax.numpy as jnp
from jax import lax
from jax.experimental import pallas as pl
from jax.experimental.pallas import tpu as pltpu
from jax.experimental.pallas import tpu_sc as plsc

N_NODES = 10000
N_EDGES = 320000
H = 128
HEADS = 8
DH = 16

_BN_SCALE = 1.0 / (1.0 + 1e-5) ** 0.5

ROWS_BLK = 2000

_NC = 2
_NS = 16
_NW = _NC * _NS
_EPW = N_EDGES // _NW
_BE = 80
_NB = _EPW // _BE
_NG = 5
_NBG = 25
_RZ = 1000
_NZ = N_NODES // _RZ

_sc_mesh = plsc.VectorSubcoreMesh(core_axis_name="c", subcore_axis_name="s")


def _sc_deg_body(dst_hbm, ones_hbm, zeros_hbm, out_hbm, idx_v, ones_v, acc_sh):
    c = lax.axis_index("c")
    s = lax.axis_index("s")
    w = s * _NC + c

    @pl.when(s < _NZ)
    def _():
        pltpu.sync_copy(zeros_hbm, acc_sh.at[pl.ds(s * _RZ, _RZ)])

    pltpu.sync_copy(ones_hbm, ones_v)
    pltpu.sync_copy(dst_hbm.at[w], idx_v)
    plsc.subcore_barrier()

    def step(i, carry):
        pltpu.sync_copy(ones_v, acc_sh.at[idx_v.at[i]], add=True)
        return carry

    lax.fori_loop(0, _NB, step, 0)
    plsc.subcore_barrier()

    @pl.when(s < _NZ)
    def _():
        pltpu.sync_copy(acc_sh.at[pl.ds(s * _RZ, _RZ)],
                        out_hbm.at[c, pl.ds(s * _RZ, _RZ)])


def _sc_deg(dst_r):
    ones = jnp.ones((_BE, H), jnp.float32)
    zeros = jnp.zeros((_RZ, H), jnp.float32)
    return pl.kernel(
        _sc_deg_body,
        name="sc_deg",
        out_type=jax.ShapeDtypeStruct((_NC, N_NODES, H), jnp.float32),
        mesh=_sc_mesh,
        scratch_types=[
            pltpu.VMEM((_NB, _BE), jnp.int32),
            pltpu.VMEM((_BE, H), jnp.float32),
            pltpu.VMEM_SHARED((N_NODES, H), jnp.float32),
        ],
    )(dst_r, ones, zeros)


def _sc_gather_add_body(tab_hbm, src_hbm, dst_hbm, zeros_hbm, out_hbm,
                        sidx_v, didx_v, rows_v, acc_sh, sem):
    c = lax.axis_index("c")
    s = lax.axis_index("s")
    w = s * _NC + c

    @pl.when(s < _NZ)
    def _():
        pltpu.sync_copy(zeros_hbm, acc_sh.at[pl.ds(s * _RZ, _RZ)])

    pltpu.sync_copy(src_hbm.at[w], sidx_v)
    pltpu.sync_copy(dst_hbm.at[w], didx_v)
    plsc.subcore_barrier()

    def step(i, carry):
        pltpu.async_copy(tab_hbm.at[sidx_v.at[i]], rows_v, sem).wait()
        pltpu.sync_copy(rows_v, acc_sh.at[didx_v.at[i]], add=True)
        return carry

    lax.fori_loop(0, _NB, step, 0)
    plsc.subcore_barrier()

    @pl.when(s < _NZ)
    def _():
        pltpu.sync_copy(acc_sh.at[pl.ds(s * _RZ, _RZ)],
                        out_hbm.at[c, pl.ds(s * _RZ, _RZ)])


def _sc_gather_add(tab, src_r, dst_r):
    zeros = jnp.zeros((_RZ, H), jnp.float32)
    return pl.kernel(
        _sc_gather_add_body,
        name="sc_gcn",
        out_type=jax.ShapeDtypeStruct((_NC, N_NODES, H), jnp.float32),
        mesh=_sc_mesh,
        scratch_types=[
            pltpu.VMEM((_NB, _BE), jnp.int32),
            pltpu.VMEM((_NB, _BE), jnp.int32),
            pltpu.VMEM((_BE, H), jnp.float32),
            pltpu.VMEM_SHARED((N_NODES, H), jnp.float32),
            pltpu.SemaphoreType.DMA,
        ],
    )(tab, src_r, dst_r, zeros)


def _sc_gat1_body(asrc_hbm, adst_hbm, src_hbm, dst_hbm, gmax_hbm, zeros_hbm,
                  den_hbm, ee_hbm, sidx_v, didx_v, arow_v, brow_v,
                  eepk_v, gm_v, den_sh, sem):
    c = lax.axis_index("c")
    s = lax.axis_index("s")
    w = s * _NC + c

    @pl.when(s < _NZ)
    def _():
        pltpu.sync_copy(zeros_hbm, den_sh.at[pl.ds(s * _RZ, _RZ)])

    pltpu.sync_copy(gmax_hbm, gm_v)
    plsc.subcore_barrier()
    gv = gm_v[0, pl.ds(0, 16)]

    def group(g, carry):
        pltpu.sync_copy(src_hbm.at[w, g], sidx_v)
        pltpu.sync_copy(dst_hbm.at[w, g], didx_v)

        def step(i, carry1):
            pltpu.async_copy(asrc_hbm.at[sidx_v.at[i]], arow_v, sem).wait()
            pltpu.async_copy(adst_hbm.at[didx_v.at[i]], brow_v, sem).wait()

            def edge(j, carry2):
                e = arow_v[j, pl.ds(0, 16)] + brow_v[j, pl.ds(0, 16)]
                e = jnp.maximum(e, 0.2 * e)
                ee = jnp.exp(e - gv)
                brow_v[j, pl.ds(0, 16)] = ee
                eepk_v[j // 8, pl.ds((j % 8) * 16, 16)] = ee
                return carry2

            lax.fori_loop(0, _BE, edge, 0)
            pltpu.sync_copy(brow_v, den_sh.at[didx_v.at[i]], add=True)
            pltpu.sync_copy(eepk_v, ee_hbm.at[w * _NB + g * _NBG + i])
            return carry1

        lax.fori_loop(0, _NBG, step, 0)
        return carry

    lax.fori_loop(0, _NG, group, 0)
    plsc.subcore_barrier()

    @pl.when(s < _NZ)
    def _():
        pltpu.sync_copy(den_sh.at[pl.ds(s * _RZ, _RZ)],
                        den_hbm.at[c, pl.ds(s * _RZ, _RZ)])


def _sc_gat1(asrc128, adst128, src_r4, dst_r4, gmax_arr):
    zeros = jnp.zeros((_RZ, H), jnp.float32)
    return pl.kernel(
        _sc_gat1_body,
        name="sc_gat1",
        out_type=[
            jax.ShapeDtypeStruct((_NC, N_NODES, H), jnp.float32),
            jax.ShapeDtypeStruct((_NW * _NB, _BE // 8, H), jnp.float32),
        ],
        mesh=_sc_mesh,
        scratch_types=[
            pltpu.VMEM((_NBG, _BE), jnp.int32),
            pltpu.VMEM((_NBG, _BE), jnp.int32),
            pltpu.VMEM((_BE, H), jnp.float32),
            pltpu.VMEM((_BE, H), jnp.float32),
            pltpu.VMEM((_BE // 8, H), jnp.float32),
            pltpu.VMEM((8, H), jnp.float32),
            pltpu.VMEM_SHARED((N_NODES, H), jnp.float32),
            pltpu.SemaphoreType.DMA,
        ],
    )(asrc128, adst128, src_r4, dst_r4, gmax_arr, zeros)


def _sc_gat2_body(hg_hbm, ee_hbm, src_hbm, dst_hbm, zeros_hbm, out_hbm,
                  sidx_v, didx_v, rows_v, ee_v, acc_sh, sem):
    c = lax.axis_index("c")
    s = lax.axis_index("s")
    w = s * _NC + c

    @pl.when(s < _NZ)
    def _():
        pltpu.sync_copy(zeros_hbm, acc_sh.at[pl.ds(s * _RZ, _RZ)])

    plsc.subcore_barrier()

    def group(g, carry):
        pltpu.sync_copy(src_hbm.at[w, g], sidx_v)
        pltpu.sync_copy(dst_hbm.at[w, g], didx_v)

        def step(i, carry1):
            pltpu.async_copy(hg_hbm.at[sidx_v.at[i]], rows_v, sem).wait()
            pltpu.sync_copy(ee_hbm.at[w * _NB + g * _NBG + i], ee_v)

            def edge(j, carry2):
                eev = ee_v[j // 8, pl.ds((j % 8) * 16, 16)]
                for k in range(HEADS):
                    rows_v[j, pl.ds(k * 16, 16)] = (
                        rows_v[j, pl.ds(k * 16, 16)] * eev[k])
                return carry2

            lax.fori_loop(0, _BE, edge, 0)
            pltpu.sync_copy(rows_v, acc_sh.at[didx_v.at[i]], add=True)
            return carry1

        lax.fori_loop(0, _NBG, step, 0)
        return carry

    lax.fori_loop(0, _NG, group, 0)
    plsc.subcore_barrier()

    @pl.when(s < _NZ)
    def _():
        pltpu.sync_copy(acc_sh.at[pl.ds(s * _RZ, _RZ)],
                        out_hbm.at[c, pl.ds(s * _RZ, _RZ)])


def _sc_gat2(hg, eebuf, src_r4, dst_r4):
    zeros = jnp.zeros((_RZ, H), jnp.float32)
    return pl.kernel(
        _sc_gat2_body,
        name="sc_gat2",
        out_type=jax.ShapeDtypeStruct((_NC, N_NODES, H), jnp.float32),
        mesh=_sc_mesh,
        scratch_types=[
            pltpu.VMEM((_NBG, _BE), jnp.int32),
            pltpu.VMEM((_NBG, _BE), jnp.int32),
            pltpu.VMEM((_BE, H), jnp.float32),
            pltpu.VMEM((_BE // 8, H), jnp.float32),
            pltpu.VMEM_SHARED((N_NODES, H), jnp.float32),
            pltpu.SemaphoreType.DMA,
        ],
    )(hg, eebuf, src_r4, dst_r4, zeros)


def _prelude_body(x_ref, w_ref, b_ref, g_ref, bb_ref, gw_ref, degp_ref,
                  x1_ref, hp_ref):
    h = jnp.dot(x_ref[...], w_ref[...], preferred_element_type=jnp.float32)
    h = (h + b_ref[...]) * (_BN_SCALE * g_ref[...]) + bb_ref[...]
    x1 = jnp.maximum(h, 0.0)
    x1_ref[...] = x1
    d = degp_ref[...]
    deg = d[0, :, 0:1] + d[1, :, 0:1] + 1.0
    dinv = lax.rsqrt(deg)
    hp_ref[...] = jnp.dot(x1, gw_ref[...],
                          preferred_element_type=jnp.float32) * dinv


def _prelude(x, w, b, g, bb, gw, degp):
    grid = (N_NODES // ROWS_BLK,)
    return pl.pallas_call(
        _prelude_body,
        grid=grid,
        in_specs=[
            pl.BlockSpec((ROWS_BLK, H), lambda i: (i, 0)),
            pl.BlockSpec((H, H), lambda i: (0, 0)),
            pl.BlockSpec((1, H), lambda i: (0, 0)),
            pl.BlockSpec((1, H), lambda i: (0, 0)),
            pl.BlockSpec((1, H), lambda i: (0, 0)),
            pl.BlockSpec((H, H), lambda i: (0, 0)),
            pl.BlockSpec((_NC, ROWS_BLK, H), lambda i: (0, i, 0)),
        ],
        out_specs=[
            pl.BlockSpec((ROWS_BLK, H), lambda i: (i, 0)),
            pl.BlockSpec((ROWS_BLK, H), lambda i: (i, 0)),
        ],
        out_shape=[
            jax.ShapeDtypeStruct((N_NODES, H), jnp.float32),
            jax.ShapeDtypeStruct((N_NODES, H), jnp.float32),
        ],
    )(x, w, b.reshape(1, H), g.reshape(1, H), bb.reshape(1, H), gw, degp)


def _ln(h, g, b):
    m = h.mean(-1, keepdims=True)
    v = h.var(-1, keepdims=True)
    return (h - m) / jnp.sqrt(v + 1e-5) * g + b


def _bn(h, g, b):
    return h * (_BN_SCALE * g) + b


def _gat_sc(x, src_r4, dst_r4, W, a_s, a_d, b):
    N = x.shape[0]
    hg = x @ W
    hr = hg.reshape(N, HEADS, DH)
    als = (hr * a_s[None]).sum(-1)
    ald = (hr * a_d[None]).sum(-1)
    gmax8 = jax.nn.leaky_relu(als.max(0) + ald.max(0), 0.2)
    asrc128 = jnp.full((N, H), -1e30, jnp.float32).at[:, :HEADS].set(als)
    adst128 = jnp.full((N, H), -1e30, jnp.float32).at[:, :HEADS].set(ald)
    gmax_arr = jnp.zeros((8, H), jnp.float32).at[0, :HEADS].set(gmax8)
    denp, eebuf = _sc_gat1(asrc128, adst128, src_r4, dst_r4, gmax_arr)
    accp = _sc_gat2(hg, eebuf, src_r4, dst_r4)
    eel = jnp.exp(jax.nn.leaky_relu(als + ald, 0.2) - gmax8[None])
    den = denp[0, :, :HEADS] + denp[1, :, :HEADS] + eel
    accr = (accp[0] + accp[1]).reshape(N, HEADS, DH) + eel[:, :, None] * hr
    out = accr / (den[:, :, None] + 1e-16)
    return out.reshape(N, H) + b


def kernel(x, edge_index, params):
    P = params
    src = edge_index[0]
    dst = edge_index[1]
    src_r = src.reshape(_NW, _NB, _BE).astype(jnp.int32)
    dst_r = dst.reshape(_NW, _NB, _BE).astype(jnp.int32)
    src_r4 = src_r.reshape(_NW, _NG, _NBG, _BE)
    dst_r4 = dst_r.reshape(_NW, _NG, _NBG, _BE)

    degp = _sc_deg(dst_r)
    x1, hp = _prelude(x, P['np_W'], P['np_b'], P['np_bn_g'], P['np_bn_b'],
                      P['gcn_W'], degp)
    acc = _sc_gather_add(hp, src_r, dst_r)

    deg = degp[0, :, 0] + degp[1, :, 0] + 1.0
    dinv = lax.rsqrt(deg)[:, None]
    h = dinv * (acc[0] + acc[1] + hp) + P['gcn_b']
    h = _ln(h, P['ln0_g'], P['ln0_b']) + x1
    xx = jax.nn.relu(h)

    for i in range(1, 4):
        h = _gat_sc(xx, src_r4, dst_r4, P['gat%d_W' % i], P['gat%d_as' % i],
                    P['gat%d_ad' % i], P['gat%d_b' % i])
        h = _ln(h, P['ln%d_g' % i], P['ln%d_b' % i]) + xx
        xx = jax.nn.relu(h)
    pw = jax.nn.softmax(P['pool_w'])
    gm = xx.mean(0, keepdims=True) * pw[0]
    gx = xx.max(0, keepdims=True) * pw[1]
    ga = xx.sum(0, keepdims=True) * pw[2]
    gf = jnp.concatenate([gm, gx, ga], axis=1)
    ctx = xx.mean(0, keepdims=True)
    ctx = jax.nn.relu(ctx @ P['c1_W'] + P['c1_b'])
    ctx = ctx @ P['c2_W'] + P['c2_b']
    ctx = jnp.broadcast_to(ctx, (xx.shape[0], ctx.shape[1]))
    nwc = jnp.concatenate([xx, ctx], axis=1)
    s = jax.nn.relu(_bn(nwc @ P['s1_W'] + P['s1_b'], P['s1_bn_g'], P['s1_bn_b']))
    s = jax.nn.relu(s @ P['s2_W'] + P['s2_b'])
    s = s @ P['s3_W'] + P['s3_b']
    scores = jax.nn.sigmoid(s[:, 0])
    c = jax.nn.relu(_bn(gf @ P['h1_W'] + P['h1_b'], P['h1_bn_g'], P['h1_bn_b']))
    c = jax.nn.relu(_bn(c @ P['h2_W'] + P['h2_b'], P['h2_bn_g'], P['h2_bn_b']))
    c = jax.nn.relu(c @ P['h3_W'] + P['h3_b'])
    logits = c @ P['h4_W'] + P['h4_b']
    fp = jax.nn.relu(xx @ P['p1_W'] + P['p1_b'])
    fp = fp @ P['p2_W'] + P['p2_b']
    return logits, scores, gf, xx, fp
